# baseline (speedup 1.0000x reference)
import jax
import jax.numpy as jnp
from jax.experimental import pallas as pl
from jax.experimental.pallas import tpu as pltpu

_K_NN = 5


def _nn_body(lab_ref, q_ref, queue_ref, out_ref):
    tile = queue_ref[0]
    qv = q_ref[0]
    dot = jax.lax.dot_general(qv, tile, (((1,), (0,)), ((), ())),
                              precision=jax.lax.Precision.HIGHEST,
                              preferred_element_type=jnp.float32)
    nrm2 = jnp.sum(tile * tile, axis=0, keepdims=True)
    inv = 1.0 / (jnp.sqrt(nrm2) + 1e-12)
    dist = nrm2 * inv * inv - 2.0 * dot * inv
    s = dist.shape[1]
    idx = jax.lax.broadcasted_iota(jnp.int32, (1, s), 1)
    kth = jnp.int32(0)
    d = dist
    for _ in range(_K_NN):
        m = jnp.min(d)
        kth = jnp.min(jnp.where(d <= m, idx, s))
        d = jnp.where(idx == kth, jnp.inf, d)
    onehot = (idx == kth).astype(jnp.float32)
    rep = jax.lax.dot_general(onehot, tile, (((1,), (1,)), ((), ())),
                              precision=jax.lax.Precision.HIGHEST,
                              preferred_element_type=jnp.float32)
    inv_k = jnp.sum(onehot * inv)
    out_ref[0] = rep * inv_k


def kernel(q, labels, queue):
    n, d = q.shape
    _, _, s = queue.shape
    m = n // 2
    lab = labels[:m].astype(jnp.int32)
    perm = jnp.argsort(lab)
    lab_s = lab[perm]
    q_s = q[perm].reshape(m, 1, d)
    rep_s = pl.pallas_call(
        _nn_body,
        grid_spec=pltpu.PrefetchScalarGridSpec(
            num_scalar_prefetch=1,
            grid=(m,),
            in_specs=[
                pl.BlockSpec((1, 1, d), lambda i, lr: (i, 0, 0)),
                pl.BlockSpec((1, d, s), lambda i, lr: (lr[i], 0, 0)),
            ],
            out_specs=pl.BlockSpec((1, 1, d), lambda i, lr: (i, 0, 0)),
        ),
        out_shape=jax.ShapeDtypeStruct((m, 1, d), jnp.float32),
    )(lab_s, q_s, queue)
    return q.at[perm].set(rep_s.reshape(m, d))


# speedup vs baseline: 1.3424x; 1.3424x over previous
import jax
import jax.numpy as jnp
from jax.experimental import pallas as pl
from jax.experimental.pallas import tpu as pltpu

_K_NN = 5
_W = 8


def _nn_body(lab_ref, q_ref, *rest):
    queue_refs = rest[:_W]
    out_ref = rest[_W]
    for j in range(_W):
        tile = queue_refs[j][0]
        qv = q_ref[j, 0]
        dot = jax.lax.dot_general(qv, tile, (((1,), (0,)), ((), ())),
                                  precision=jax.lax.Precision.HIGHEST,
                                  preferred_element_type=jnp.float32)
        nrm2 = jnp.sum(tile * tile, axis=0, keepdims=True)
        inv = 1.0 / (jnp.sqrt(nrm2) + 1e-12)
        dist = nrm2 * inv * inv - 2.0 * dot * inv
        s = dist.shape[1]
        idx = jax.lax.broadcasted_iota(jnp.int32, (1, s), 1)
        kth = jnp.int32(0)
        d = dist
        for _ in range(_K_NN):
            m = jnp.min(d)
            kth = jnp.min(jnp.where(d <= m, idx, s))
            d = jnp.where(idx == kth, jnp.inf, d)
        onehot = (idx == kth).astype(jnp.float32)
        rep = jax.lax.dot_general(onehot, tile, (((1,), (1,)), ((), ())),
                                  precision=jax.lax.Precision.HIGHEST,
                                  preferred_element_type=jnp.float32)
        inv_k = jnp.sum(onehot * inv)
        out_ref[j, 0] = rep * inv_k


def kernel(q, labels, queue):
    n, d = q.shape
    _, _, s = queue.shape
    m = n // 2
    rows = m // _W
    lab = labels[:m].astype(jnp.int32)
    perm = jnp.argsort(lab)
    lab_s = lab[perm]
    q_s = q[perm].reshape(_W, rows, 1, d)

    def q_map(i, lr):
        return (0, i, 0, 0)

    def queue_map(j):
        def f(i, lr):
            return (lr[j * rows + i], 0, 0)
        return f

    rep_s = pl.pallas_call(
        _nn_body,
        grid_spec=pltpu.PrefetchScalarGridSpec(
            num_scalar_prefetch=1,
            grid=(rows,),
            in_specs=[pl.BlockSpec((_W, 1, 1, d), q_map)] +
                     [pl.BlockSpec((1, d, s), queue_map(j)) for j in range(_W)],
            out_specs=pl.BlockSpec((_W, 1, 1, d), q_map),
        ),
        out_shape=jax.ShapeDtypeStruct((_W, rows, 1, d), jnp.float32),
    )(lab_s, q_s, *([queue] * _W))
    return q.at[perm].set(rep_s.reshape(m, d))


# speedup vs baseline: 4.7940x; 3.5711x over previous
import jax
import jax.numpy as jnp
from jax.experimental import pallas as pl
from jax.experimental.pallas import tpu as pltpu

_K_NN = 5
_W = 8


def _nn_body(lab_ref, q_ref, *rest):
    queue_refs = rest[:_W]
    out_ref = rest[_W]
    dots, nrms = [], []
    for j in range(_W):
        tile = queue_refs[j][0]
        qv = q_ref[j, 0]
        dots.append(jax.lax.dot_general(
            qv, tile, (((1,), (0,)), ((), ())),
            precision=jax.lax.Precision.HIGHEST,
            preferred_element_type=jnp.float32))
        nrms.append(jnp.sum(tile * tile, axis=0, keepdims=True))
    dot8 = jnp.concatenate(dots, axis=0)
    nrm8 = jnp.concatenate(nrms, axis=0)
    inv8 = 1.0 / (jnp.sqrt(nrm8) + 1e-12)
    dist8 = nrm8 * inv8 * inv8 - 2.0 * dot8 * inv8
    s = dist8.shape[1]
    idx = jax.lax.broadcasted_iota(jnp.int32, (_W, s), 1)
    kth = None
    d = dist8
    for _ in range(_K_NN):
        mn = jnp.min(d, axis=1, keepdims=True)
        kth = jnp.min(jnp.where(d <= mn, idx, s), axis=1, keepdims=True)
        d = jnp.where(idx == kth, jnp.inf, d)
    onehot8 = (idx == kth).astype(jnp.float32)
    invk8 = jnp.sum(onehot8 * inv8, axis=1, keepdims=True)
    for j in range(_W):
        tile = queue_refs[j][0]
        rep = jax.lax.dot_general(onehot8[j:j + 1], tile,
                                  (((1,), (1,)), ((), ())),
                                  preferred_element_type=jnp.float32)
        out_ref[j, 0] = rep * invk8[j:j + 1]


def kernel(q, labels, queue):
    n, d = q.shape
    _, _, s = queue.shape
    m = n // 2
    rows = m // _W
    lab = labels[:m].astype(jnp.int32)
    perm = jnp.argsort(lab)
    lab_s = lab[perm]
    q_s = q[perm].reshape(_W, rows, 1, d)

    def q_map(i, lr):
        return (0, i, 0, 0)

    def queue_map(j):
        def f(i, lr):
            return (lr[j * rows + i], 0, 0)
        return f

    rep_s = pl.pallas_call(
        _nn_body,
        grid_spec=pltpu.PrefetchScalarGridSpec(
            num_scalar_prefetch=1,
            grid=(rows,),
            in_specs=[pl.BlockSpec((_W, 1, 1, d), q_map)] +
                     [pl.BlockSpec((1, d, s), queue_map(j)) for j in range(_W)],
            out_specs=pl.BlockSpec((_W, 1, 1, d), q_map),
        ),
        out_shape=jax.ShapeDtypeStruct((_W, rows, 1, d), jnp.float32),
    )(lab_s, q_s, *([queue] * _W))
    return q.at[perm].set(rep_s.reshape(m, d))
